# baseline (speedup 1.0000x reference)
import jax
import jax.numpy as jnp
from jax import lax
from jax.experimental import pallas as pl
from jax.experimental.pallas import tpu as pltpu
from jax.experimental.pallas import tpu_sc as plsc

B, S, D = 16384, 100, 64
K = 26
NC, NS, L = 2, 16, 16
NW = NC * NS
ROWS = B * K
RPW = ROWS // NW
CHUNK = 104
NCH = RPW // CHUNK


def _body(x_ref, idx_ref, out_ref, idx26_v, idx_all, rows_v, sem):
    wid = lax.axis_index("s") * NC + lax.axis_index("c")
    row0 = wid * RPW
    b0 = wid * (B // NW)
    pltpu.sync_copy(idx_ref, idx26_v)
    ga = idx26_v[pl.ds(0, L)]
    gb = idx26_v[pl.ds(K - L, L)]

    @pl.loop(0, B // NW, unroll=4)
    def _build(lb):
        base = (b0 + lb) * S
        o = lb * K
        idx_all[pl.ds(o, L)] = base + ga
        idx_all[pl.ds(o + (K - L), L)] = base + gb

    @pl.loop(0, NCH)
    def _chunk(c):
        off = pl.multiple_of(c * CHUNK, CHUNK)
        pltpu.async_copy(x_ref.at[idx_all.at[pl.ds(off, CHUNK)]], rows_v,
                         sem).wait()
        pltpu.sync_copy(rows_v, out_ref.at[pl.ds(row0 + off, CHUNK)])


def _gather_rows(x2d, indices):
    mesh = plsc.VectorSubcoreMesh(core_axis_name="c", subcore_axis_name="s",
                                  num_cores=NC, num_subcores=NS)
    return pl.kernel(
        _body,
        out_type=jax.ShapeDtypeStruct((ROWS, D), jnp.float32),
        mesh=mesh,
        compiler_params=pltpu.CompilerParams(use_tc_tiling_on_sc=False),
        scratch_types=[
            pltpu.VMEM((K,), jnp.int32),
            pltpu.VMEM((RPW,), jnp.int32),
            pltpu.VMEM((CHUNK, D), jnp.float32),
            pltpu.SemaphoreType.DMA,
        ],
    )(x2d, indices)


def kernel(x, indices):
    out = _gather_rows(x.reshape(B * S, D), indices)
    return out.reshape(B, K, D)


# speedup vs baseline: 1.0712x; 1.0712x over previous
import jax
import jax.numpy as jnp
from jax import lax
from jax.experimental import pallas as pl
from jax.experimental.pallas import tpu as pltpu
from jax.experimental.pallas import tpu_sc as plsc

B, S, D = 16384, 100, 64
K = 26
NC, NS, L = 2, 16, 16
NW = NC * NS
ROWS = B * K
RPW = ROWS // NW
BPW = B // NW
CHUNK = 104
GPS = 4
SCH = GPS * CHUNK
NSC = RPW // SCH
NBUF = 4
LEAD = 2


def _body(x_ref, idx_ref, out_ref, idx26_v, idx_all,
          rv0, rv1, rv2, rv3, g0, g1, g2, g3, s0, s1, s2, s3):
    rows_v = (rv0, rv1, rv2, rv3)
    gsem = (g0, g1, g2, g3)
    ssem = (s0, s1, s2, s3)
    wid = lax.axis_index("s") * NC + lax.axis_index("c")
    row0 = wid * RPW
    b0 = wid * BPW
    pltpu.sync_copy(idx_ref, idx26_v)
    ga = idx26_v[pl.ds(0, L)]
    gb = idx26_v[pl.ds(K - L, L)]

    @pl.loop(0, BPW, unroll=4)
    def _build(lb):
        base = (b0 + lb) * S
        o = lb * K
        idx_all[pl.ds(o, L)] = base + ga
        idx_all[pl.ds(o + (K - L), L)] = base + gb

    def start_gather(sc):
        slot = sc % NBUF
        for g in range(GPS):
            off = sc * SCH + g * CHUNK
            pltpu.async_copy(x_ref.at[idx_all.at[pl.ds(off, CHUNK)]],
                             rows_v[slot].at[pl.ds(g * CHUNK, CHUNK)],
                             gsem[slot])

    def wait_gather(sc):
        slot = sc % NBUF
        for g in range(GPS):
            pltpu.make_async_copy(
                x_ref.at[idx_all.at[pl.ds(sc * SCH + g * CHUNK, CHUNK)]],
                rows_v[slot].at[pl.ds(g * CHUNK, CHUNK)],
                gsem[slot]).wait()

    def store_dst(sc):
        return out_ref.at[pl.ds(row0 + sc * SCH, SCH)]

    for sc in range(LEAD):
        start_gather(sc)
    for sc in range(NSC):
        slot = sc % NBUF
        nxt = sc + LEAD
        if nxt < NSC:
            nslot = nxt % NBUF
            if nxt - NBUF >= 0:
                pltpu.make_async_copy(rows_v[nslot], store_dst(nxt - NBUF),
                                      ssem[nslot]).wait()
            start_gather(nxt)
        wait_gather(sc)
        pltpu.async_copy(rows_v[slot], store_dst(sc), ssem[slot])
    for sc in range(NSC - NBUF, NSC):
        slot = sc % NBUF
        pltpu.make_async_copy(rows_v[slot], store_dst(sc), ssem[slot]).wait()


def _gather_rows(x2d, indices):
    mesh = plsc.VectorSubcoreMesh(core_axis_name="c", subcore_axis_name="s",
                                  num_cores=NC, num_subcores=NS)
    return pl.kernel(
        _body,
        out_type=jax.ShapeDtypeStruct((ROWS, D), jnp.float32),
        mesh=mesh,
        compiler_params=pltpu.CompilerParams(use_tc_tiling_on_sc=False),
        scratch_types=[
            pltpu.VMEM((K,), jnp.int32),
            pltpu.VMEM((RPW,), jnp.int32),
        ] + [pltpu.VMEM((SCH, D), jnp.float32) for _ in range(NBUF)]
          + [pltpu.SemaphoreType.DMA for _ in range(2 * NBUF)],
    )(x2d, indices)


def kernel(x, indices):
    out = _gather_rows(x.reshape(B * S, D), indices)
    return out.reshape(B, K, D)
